# baseline (speedup 1.0000x reference)
import functools

import jax
import jax.numpy as jnp
from jax import lax
from jax.experimental import pallas as pl
from jax.experimental.pallas import tpu as pltpu
from jax.experimental.pallas import tpu_sc as plsc

NC, NS, L = 2, 16, 16
NW = NC * NS

K = 64
V = 513
D = 768
R = 512
RPW = R // NW
IPW = RPW * K
NV = D // L

_mesh = plsc.VectorSubcoreMesh(core_axis_name="c", subcore_axis_name="s")


@functools.partial(
    pl.kernel,
    out_type=jax.ShapeDtypeStruct((R, D), jnp.float32),
    mesh=_mesh,
    scratch_types=[
        pltpu.VMEM((IPW,), jnp.int32),
        pltpu.VMEM((IPW,), jnp.int32),
        pltpu.VMEM((RPW, D), jnp.float32),
        pltpu.VMEM((RPW, D), jnp.float32),
        pltpu.VMEM((RPW, D), jnp.float32),
        pltpu.SemaphoreType.DMA,
        pltpu.SemaphoreType.DMA,
    ],
    compiler_params=pltpu.CompilerParams(use_tc_tiling_on_sc=True, needs_layout_passes=False),
)
def _emb_sum(ids_hbm, table_hbm, out_hbm, idx_v, idxt_v, rows_a, rows_b,
             acc_v, sem_a, sem_b):
    wid = lax.axis_index("s") * NC + lax.axis_index("c")
    base = wid * IPW
    pltpu.sync_copy(ids_hbm.at[pl.ds(base, IPW)], idx_v)

    lane64 = lax.iota(jnp.int32, L) * K

    def transpose(i, _):
        vec = plsc.load_gather(idx_v, [lane64 + i])
        idxt_v[pl.ds(i * L, L)] = vec
        return 0

    lax.fori_loop(0, K, transpose, 0)

    zero = jnp.zeros((L,), jnp.float32)

    def clear(j, _):
        def clear_c(c, _):
            acc_v[j, pl.ds(c * L, L)] = zero
            return 0

        lax.fori_loop(0, NV, clear_c, 0, unroll=8)
        return 0

    lax.fori_loop(0, RPW, clear, 0)

    def gather(i, buf, sem):
        return pltpu.make_async_copy(
            table_hbm.at[i].at[idxt_v.at[pl.ds(i * L, L)]], buf, sem)

    def accum(buf):
        def row(j, _):
            def body(c, _):
                plsc.addupdate(acc_v.at[j, pl.ds(c * L, L)],
                               buf[j, pl.ds(c * L, L)])
                return 0

            lax.fori_loop(0, NV, body, 0, unroll=8)
            return 0

        lax.fori_loop(0, RPW, row, 0)

    gather(0, rows_a, sem_a).start()

    def pair(h, _):
        i0 = 2 * h
        gather(i0 + 1, rows_b, sem_b).start()
        gather(i0, rows_a, sem_a).wait()
        accum(rows_a)

        @pl.when(i0 + 2 < K)
        def _():
            gather(i0 + 2, rows_a, sem_a).start()

        gather(i0 + 1, rows_b, sem_b).wait()
        accum(rows_b)
        return 0

    lax.fori_loop(0, K // 2, pair, 0)
    pltpu.sync_copy(acc_v, out_hbm.at[pl.ds(wid * RPW, RPW)])


def kernel(input_ids, tables):
    b, seq = input_ids.shape
    ids = input_ids.astype(jnp.int32).reshape(-1)
    out = _emb_sum(ids, tables)
    return out.reshape(b, seq // K, D)


# speedup vs baseline: 1.4099x; 1.4099x over previous
import functools

import jax
import jax.numpy as jnp
from jax import lax
from jax.experimental import pallas as pl
from jax.experimental.pallas import tpu as pltpu
from jax.experimental.pallas import tpu_sc as plsc

NC, NS, L = 2, 16, 16
NW = NC * NS

K = 64
V = 513
D = 768
R = 512
RPW = R // NW
IPW = RPW * K
NV = D // L

_mesh = plsc.VectorSubcoreMesh(core_axis_name="c", subcore_axis_name="s")


@functools.partial(
    pl.kernel,
    out_type=jax.ShapeDtypeStruct((R, D), jnp.float32),
    mesh=_mesh,
    scratch_types=[
        pltpu.VMEM((IPW,), jnp.int32),
        pltpu.VMEM((IPW,), jnp.int32),
        pltpu.VMEM((RPW, D), jnp.float32),
        pltpu.VMEM((RPW, D), jnp.float32),
        pltpu.VMEM((RPW, D), jnp.float32),
        pltpu.SemaphoreType.DMA,
        pltpu.SemaphoreType.DMA,
    ],
    compiler_params=pltpu.CompilerParams(use_tc_tiling_on_sc=True, needs_layout_passes=False),
)
def _emb_sum(ids_hbm, table_hbm, out_hbm, idx_v, idxt_v, rows_a, rows_b,
             acc_v, sem_a, sem_b):
    wid = lax.axis_index("s") * NC + lax.axis_index("c")
    base = wid * IPW
    pltpu.sync_copy(ids_hbm.at[pl.ds(base, IPW)], idx_v)

    lane64 = lax.iota(jnp.int32, L) * K

    def transpose(i, _):
        vec = plsc.load_gather(idx_v, [lane64 + i])
        idxt_v[pl.ds(i * L, L)] = vec
        return 0

    lax.fori_loop(0, K, transpose, 0)

    zero = jnp.zeros((L,), jnp.float32)

    def clear(c, _):
        for j in range(RPW):
            acc_v[j, pl.ds(c * L, L)] = zero
        return 0

    lax.fori_loop(0, NV, clear, 0)

    def gather(i, buf, sem):
        return pltpu.make_async_copy(
            table_hbm.at[i].at[idxt_v.at[pl.ds(i * L, L)]], buf, sem)

    def accum(buf):
        def body(c, _):
            for j in range(RPW):
                plsc.addupdate(acc_v.at[j, pl.ds(c * L, L)],
                               buf[j, pl.ds(c * L, L)])
            return 0

        lax.fori_loop(0, NV, body, 0)

    gather(0, rows_a, sem_a).start()

    def pair(h, _):
        i0 = 2 * h
        gather(i0 + 1, rows_b, sem_b).start()
        gather(i0, rows_a, sem_a).wait()
        accum(rows_a)

        @pl.when(i0 + 2 < K)
        def _():
            gather(i0 + 2, rows_a, sem_a).start()

        gather(i0 + 1, rows_b, sem_b).wait()
        accum(rows_b)
        return 0

    lax.fori_loop(0, K // 2, pair, 0)
    pltpu.sync_copy(acc_v, out_hbm.at[pl.ds(wid * RPW, RPW)])


def kernel(input_ids, tables):
    b, seq = input_ids.shape
    ids = input_ids.astype(jnp.int32).reshape(-1)
    out = _emb_sum(ids, tables)
    return out.reshape(b, seq // K, D)
